# Initial kernel scaffold; baseline (speedup 1.0000x reference)
import jax
import jax.numpy as jnp
from jax.experimental import pallas as pl


def kernel(x, p_l, p_r, k_l, k_r, b_g):
    raise NotImplementedError("write your pallas kernel here")


# speedup vs baseline: 1.3545x; 1.3545x over previous
import jax
import jax.numpy as jnp
from jax.experimental import pallas as pl
from jax.experimental.pallas import tpu as pltpu

_N = 4


def _gal_body(p_l, p_r, k_l, k_r, b_g, x_ref, o_ref):
    x = x_ref[...]
    acc = (
        jnp.maximum(x, 0.0) * k_r[0, 0]
        + jnp.minimum(x, 0.0) * k_l[0, 0]
        + b_g[0]
    )
    for j in range(1, _N + 1):
        acc += (k_r[j, 0] - k_r[j - 1, 0]) * jnp.maximum(x - p_r[j, 0], 0.0)
        acc += (k_l[j, 0] - k_l[j - 1, 0]) * jnp.minimum(x - p_l[j, 0], 0.0)
    o_ref[...] = acc


def kernel(x, p_l, p_r, k_l, k_r, b_g):
    orig_shape = x.shape
    x2 = x.reshape(-1, orig_shape[-1])
    rows, cols = x2.shape
    block_rows = 1024
    grid = (rows // block_rows,)

    smem = pl.BlockSpec(memory_space=pltpu.SMEM)
    out = pl.pallas_call(
        _gal_body,
        grid=grid,
        in_specs=[
            smem,
            smem,
            smem,
            smem,
            smem,
            pl.BlockSpec((block_rows, cols), lambda i: (i, 0)),
        ],
        out_specs=pl.BlockSpec((block_rows, cols), lambda i: (i, 0)),
        out_shape=jax.ShapeDtypeStruct((rows, cols), x.dtype),
    )(p_l, p_r, k_l, k_r, b_g, x2)
    return out.reshape(orig_shape)
